# baseline (device time: 16244 ns/iter reference)
import contextlib
import os

import jax
import jax.numpy as jnp
from jax import lax
from jax.experimental import pallas as pl
from jax.experimental.pallas import tpu as pltpu

N_DEV = 8
WIRE_DT = jnp.bfloat16

if os.environ.get("KERNEL_PROFILE_SCOPES") == "1":
    _scope = jax.named_scope
else:
    def _scope(name):
        return contextlib.nullcontext()


def kernel(table, idx):
    m_per, d = table.shape
    n = idx.shape[0]
    seg = n // N_DEV
    dh = d // 2

    def body(table_hbm, idx_ref, out_ref, tab_vmem, part_ref,
             seg0_ref, seg1_ref, rs0_ref, rs1_ref, ag0_ref, ag1_ref,
             rs0_send, rs0_recv, rs1_send, rs1_recv,
             ag0_send, ag0_recv, ag1_send, ag1_recv, tab_sem):
        my = lax.axis_index("i")
        halves = [
            (0, seg0_ref, rs0_ref, ag0_ref,
             rs0_send, rs0_recv, ag0_send, ag0_recv),
            (dh, seg1_ref, rs1_ref, ag1_ref,
             rs1_send, rs1_recv, ag1_send, ag1_recv),
        ]

        tcopy = pltpu.make_async_copy(table_hbm, tab_vmem, tab_sem)
        tcopy.start()

        barrier = pltpu.get_barrier_semaphore()
        with _scope("barrier_signal"):
            for k in range(1, N_DEV):
                pl.semaphore_signal(
                    barrier, inc=1,
                    device_id=(lax.rem(my + k, N_DEV),),
                    device_id_type=pl.DeviceIdType.MESH,
                )

        lo = my * m_per
        tcopy.wait()
        tab = tab_vmem[...].astype(WIRE_DT)

        chunk = 2 * seg
        my_q = lax.div(my, 2)
        sends = []
        for j in range(N_DEV // 2):
            q = lax.rem(my_q + 1 + j, N_DEV // 2)
            with _scope(f"rs_mm_{j}"):
                iv = idx_ref[pl.ds(q * chunk, chunk)][None, :] - lo
                onehot_t = (
                    iv == lax.broadcasted_iota(jnp.int32, (m_per, chunk), 0)
                ).astype(WIRE_DT)
                for c0, *_ in halves:
                    partial = lax.dot_general(
                        onehot_t, tab[:, c0:c0 + dh],
                        dimension_numbers=(((0,), (0,)), ((), ())),
                        preferred_element_type=jnp.float32,
                    )
                    part_ref[pl.ds(q * chunk, chunk), pl.ds(c0, dh)] = (
                        partial.astype(WIRE_DT))
            if j == 0:
                with _scope("barrier_wait"):
                    pl.semaphore_wait(barrier, N_DEV - 1)
            with _scope(f"rs_send_{j}"):
                for c0, _, rs_ref, _, rs_send, rs_recv, _, _ in halves:
                    for t in range(2):
                        p = q * 2 + t
                        r = pltpu.make_async_remote_copy(
                            src_ref=part_ref.at[pl.ds(p * seg, seg),
                                                pl.ds(c0, dh)],
                            dst_ref=rs_ref.at[my],
                            send_sem=rs_send.at[p],
                            recv_sem=rs_recv.at[my],
                            device_id=(p,),
                            device_id_type=pl.DeviceIdType.MESH,
                        )
                        r.start()
                        sends.append(r)

        for hi, (c0, sref, rs_ref, ag_ref,
                 rs_send, rs_recv, ag_send, ag_recv) in enumerate(halves):
            with _scope(f"rs_wait_acc_{hi}"):
                for s in range(N_DEV):
                    pltpu.make_async_remote_copy(
                        src_ref=part_ref.at[pl.ds(s * seg, seg),
                                            pl.ds(c0, dh)],
                        dst_ref=rs_ref.at[s],
                        send_sem=rs_send.at[s],
                        recv_sem=rs_recv.at[s],
                        device_id=(s,),
                        device_id_type=pl.DeviceIdType.MESH,
                    ).wait_recv()
                acc = rs_ref[0].astype(jnp.float32)
                for s in range(1, N_DEV):
                    acc = acc + rs_ref[s].astype(jnp.float32)
                sref[...] = acc.astype(WIRE_DT)
            with _scope(f"ag_issue_{hi}"):
                for k in range(1, N_DEV + 1):
                    p = lax.rem(my + k, N_DEV)
                    r = pltpu.make_async_remote_copy(
                        src_ref=sref,
                        dst_ref=ag_ref.at[my],
                        send_sem=ag_send.at[p],
                        recv_sem=ag_recv.at[my],
                        device_id=(p,),
                        device_id_type=pl.DeviceIdType.MESH,
                    )
                    r.start()
                    sends.append(r)

        for hi, (c0, sref, rs_ref, ag_ref,
                 rs_send, rs_recv, ag_send, ag_recv) in enumerate(halves):
            with _scope(f"ag_wait_store_{hi}"):
                for s in range(N_DEV):
                    pltpu.make_async_remote_copy(
                        src_ref=sref,
                        dst_ref=ag_ref.at[s],
                        send_sem=ag_send.at[s],
                        recv_sem=ag_recv.at[s],
                        device_id=(s,),
                        device_id_type=pl.DeviceIdType.MESH,
                    ).wait_recv()
                    out_ref[s * seg:(s + 1) * seg, c0:c0 + dh] = (
                        ag_ref[s].astype(jnp.float32))

        with _scope("drain_sends"):
            for r in sends:
                r.wait_send()

    return pl.pallas_call(
        body,
        out_shape=jax.ShapeDtypeStruct((n, d), jnp.float32),
        in_specs=[
            pl.BlockSpec(memory_space=pl.ANY),
            pl.BlockSpec(memory_space=pltpu.VMEM),
        ],
        out_specs=pl.BlockSpec(memory_space=pltpu.VMEM),
        scratch_shapes=[
            pltpu.VMEM((m_per, d), jnp.float32),
            pltpu.VMEM((n, d), WIRE_DT),
            pltpu.VMEM((seg, dh), WIRE_DT),
            pltpu.VMEM((seg, dh), WIRE_DT),
            pltpu.VMEM((N_DEV, seg, dh), WIRE_DT),
            pltpu.VMEM((N_DEV, seg, dh), WIRE_DT),
            pltpu.VMEM((N_DEV, seg, dh), WIRE_DT),
            pltpu.VMEM((N_DEV, seg, dh), WIRE_DT),
            pltpu.SemaphoreType.DMA((N_DEV,)),
            pltpu.SemaphoreType.DMA((N_DEV,)),
            pltpu.SemaphoreType.DMA((N_DEV,)),
            pltpu.SemaphoreType.DMA((N_DEV,)),
            pltpu.SemaphoreType.DMA((N_DEV,)),
            pltpu.SemaphoreType.DMA((N_DEV,)),
            pltpu.SemaphoreType.DMA((N_DEV,)),
            pltpu.SemaphoreType.DMA((N_DEV,)),
            pltpu.SemaphoreType.DMA,
        ],
        compiler_params=pltpu.CompilerParams(collective_id=0),
    )(table, idx)


# device time: 15515 ns/iter; 1.0470x vs baseline; 1.0470x over previous
import contextlib
import os

import jax
import jax.numpy as jnp
from jax import lax
from jax.experimental import pallas as pl
from jax.experimental.pallas import tpu as pltpu

N_DEV = 8
WIRE_DT = jnp.bfloat16

if os.environ.get("KERNEL_PROFILE_SCOPES") == "1":
    _scope = jax.named_scope
else:
    def _scope(name):
        return contextlib.nullcontext()


def kernel(table, idx):
    m_per, d = table.shape
    n = idx.shape[0]
    seg = n // N_DEV

    def body(table_hbm, idx_ref, out_ref,
             tab_vmem, part_ref, seg_ref, rs_ref, ag_ref,
             rs_send, rs_recv, ag_send, ag_recv, tab_sem):
        my = lax.axis_index("i")

        tcopy = pltpu.make_async_copy(table_hbm, tab_vmem, tab_sem)
        tcopy.start()

        barrier = pltpu.get_barrier_semaphore()
        with _scope("barrier_signal"):
            for k in range(1, N_DEV):
                pl.semaphore_signal(
                    barrier, inc=1,
                    device_id=(lax.rem(my + k, N_DEV),),
                    device_id_type=pl.DeviceIdType.MESH,
                )

        lo = my * m_per
        tcopy.wait()
        tab = tab_vmem[...].astype(WIRE_DT)

        chunk = 2 * seg
        my_q = lax.div(my, 2)
        rs_rdmas = []
        for j in range(N_DEV // 2):
            q = lax.rem(my_q + 1 + j, N_DEV // 2)
            with _scope(f"rs_mm_{j}"):
                iv = idx_ref[pl.ds(q * chunk, chunk)][None, :] - lo
                onehot_t = (
                    iv == lax.broadcasted_iota(jnp.int32, (m_per, chunk), 0)
                ).astype(WIRE_DT)
                partial = lax.dot_general(
                    onehot_t, tab,
                    dimension_numbers=(((0,), (0,)), ((), ())),
                    preferred_element_type=jnp.float32,
                )
                part_ref[pl.ds(q * chunk, chunk)] = partial.astype(WIRE_DT)
            if j == 0:
                with _scope("barrier_wait"):
                    pl.semaphore_wait(barrier, N_DEV - 1)
            with _scope(f"rs_send_{j}"):
                for t in range(2):
                    p = q * 2 + t
                    r = pltpu.make_async_remote_copy(
                        src_ref=part_ref.at[pl.ds(p * seg, seg)],
                        dst_ref=rs_ref.at[my],
                        send_sem=rs_send.at[p],
                        recv_sem=rs_recv.at[my],
                        device_id=(p,),
                        device_id_type=pl.DeviceIdType.MESH,
                    )
                    r.start()
                    rs_rdmas.append(r)

        with _scope("rs_wait_acc"):
            for s in range(N_DEV):
                pltpu.make_async_remote_copy(
                    src_ref=part_ref.at[pl.ds(s * seg, seg)],
                    dst_ref=rs_ref.at[s],
                    send_sem=rs_send.at[s],
                    recv_sem=rs_recv.at[s],
                    device_id=(s,),
                    device_id_type=pl.DeviceIdType.MESH,
                ).wait_recv()
                if s == 0:
                    acc = rs_ref[0].astype(jnp.float32)
                else:
                    acc = acc + rs_ref[s].astype(jnp.float32)
            seg_ref[...] = acc.astype(WIRE_DT)

        with _scope("ag_issue"):
            ag_rdmas = []
            for k in range(1, N_DEV + 1):
                p = lax.rem(my + k, N_DEV)
                r = pltpu.make_async_remote_copy(
                    src_ref=seg_ref,
                    dst_ref=ag_ref.at[my],
                    send_sem=ag_send.at[p],
                    recv_sem=ag_recv.at[my],
                    device_id=(p,),
                    device_id_type=pl.DeviceIdType.MESH,
                )
                r.start()
                ag_rdmas.append(r)

        with _scope("ag_wait_store"):
            for s in range(N_DEV):
                pltpu.make_async_remote_copy(
                    src_ref=seg_ref,
                    dst_ref=ag_ref.at[s],
                    send_sem=ag_send.at[s],
                    recv_sem=ag_recv.at[s],
                    device_id=(s,),
                    device_id_type=pl.DeviceIdType.MESH,
                ).wait_recv()
                out_ref[s * seg:(s + 1) * seg] = ag_ref[s]

        with _scope("drain_sends"):
            for r in rs_rdmas:
                r.wait_send()
            for r in ag_rdmas:
                r.wait_send()

    return pl.pallas_call(
        body,
        out_shape=jax.ShapeDtypeStruct((n, d), WIRE_DT),
        in_specs=[
            pl.BlockSpec(memory_space=pl.ANY),
            pl.BlockSpec(memory_space=pltpu.VMEM),
        ],
        out_specs=pl.BlockSpec(memory_space=pltpu.VMEM),
        scratch_shapes=[
            pltpu.VMEM((m_per, d), jnp.float32),
            pltpu.VMEM((n, d), WIRE_DT),
            pltpu.VMEM((seg, d), WIRE_DT),
            pltpu.VMEM((N_DEV, seg, d), WIRE_DT),
            pltpu.VMEM((N_DEV, seg, d), WIRE_DT),
            pltpu.SemaphoreType.DMA((N_DEV,)),
            pltpu.SemaphoreType.DMA((N_DEV,)),
            pltpu.SemaphoreType.DMA((N_DEV,)),
            pltpu.SemaphoreType.DMA((N_DEV,)),
            pltpu.SemaphoreType.DMA,
        ],
        compiler_params=pltpu.CompilerParams(collective_id=0),
    )(table, idx)


# device time: 15483 ns/iter; 1.0492x vs baseline; 1.0021x over previous
import contextlib
import os

import jax
import jax.numpy as jnp
from jax import lax
from jax.experimental import pallas as pl
from jax.experimental.pallas import tpu as pltpu

N_DEV = 8
WIRE_DT = jnp.bfloat16

if os.environ.get("KERNEL_PROFILE_SCOPES") == "1":
    _scope = jax.named_scope
else:
    def _scope(name):
        return contextlib.nullcontext()


def kernel(table, idx):
    m_per, d = table.shape
    n = idx.shape[0]
    seg = n // N_DEV

    def body(table_hbm, idx_ref, out_ref,
             tab_vmem, part_ref, seg_ref, rs_ref, ag_ref,
             rs_send, rs_recv, ag_send, ag_recv, tab_sem):
        my = lax.axis_index("i")

        tcopy = pltpu.make_async_copy(table_hbm, tab_vmem, tab_sem)
        tcopy.start()

        barrier = pltpu.get_barrier_semaphore()
        with _scope("barrier_signal"):
            for k in range(1, N_DEV):
                pl.semaphore_signal(
                    barrier, inc=1,
                    device_id=(lax.rem(my + k, N_DEV),),
                    device_id_type=pl.DeviceIdType.MESH,
                )

        lo = my * m_per
        tcopy.wait()
        tab = tab_vmem[...].astype(WIRE_DT)

        chunk = 2 * seg
        my_q = lax.div(my, 2)
        rs_rdmas = []
        for j in range(N_DEV // 2):
            q = lax.rem(my_q + 1 + j, N_DEV // 2)
            with _scope(f"rs_mm_{j}"):
                iv = idx_ref[pl.ds(q * chunk, chunk)][None, :] - lo
                onehot_t = (
                    iv == lax.broadcasted_iota(jnp.int32, (m_per, chunk), 0)
                ).astype(WIRE_DT)
                partial = lax.dot_general(
                    onehot_t, tab,
                    dimension_numbers=(((0,), (0,)), ((), ())),
                    preferred_element_type=jnp.float32,
                )
                part_ref[pl.ds(q * chunk, chunk)] = partial.astype(WIRE_DT)
            if j == 0:
                with _scope("barrier_wait"):
                    pl.semaphore_wait(barrier, N_DEV - 1)
            with _scope(f"rs_send_{j}"):
                if j < N_DEV // 2 - 1:
                    targets = [q * 2, q * 2 + 1]
                else:
                    pair = my + 1 - 2 * lax.rem(my, 2)
                    targets = [pair]
                    rs_ref[my] = part_ref[pl.ds(my * seg, seg)]
                for p in targets:
                    r = pltpu.make_async_remote_copy(
                        src_ref=part_ref.at[pl.ds(p * seg, seg)],
                        dst_ref=rs_ref.at[my],
                        send_sem=rs_send.at[p],
                        recv_sem=rs_recv.at[my],
                        device_id=(p,),
                        device_id_type=pl.DeviceIdType.MESH,
                    )
                    r.start()
                    rs_rdmas.append(r)

        with _scope("rs_wait_acc"):
            for k in range(1, N_DEV):
                src = lax.rem(my + k, N_DEV)
                pltpu.make_async_remote_copy(
                    src_ref=part_ref.at[pl.ds(my * seg, seg)],
                    dst_ref=rs_ref.at[src],
                    send_sem=rs_send.at[src],
                    recv_sem=rs_recv.at[src],
                    device_id=(src,),
                    device_id_type=pl.DeviceIdType.MESH,
                ).wait_recv()
            acc = rs_ref[0].astype(jnp.float32)
            for s in range(1, N_DEV):
                acc = acc + rs_ref[s].astype(jnp.float32)
            acc_w = acc.astype(WIRE_DT)
            seg_ref[...] = acc_w
            ag_ref[my] = acc_w

        with _scope("ag_issue"):
            ag_rdmas = []
            for k in range(1, N_DEV):
                p = lax.rem(my + k, N_DEV)
                r = pltpu.make_async_remote_copy(
                    src_ref=seg_ref,
                    dst_ref=ag_ref.at[my],
                    send_sem=ag_send.at[p],
                    recv_sem=ag_recv.at[my],
                    device_id=(p,),
                    device_id_type=pl.DeviceIdType.MESH,
                )
                r.start()
                ag_rdmas.append(r)

        with _scope("ag_wait_store"):
            for k in range(1, N_DEV):
                src = lax.rem(my + k, N_DEV)
                pltpu.make_async_remote_copy(
                    src_ref=seg_ref,
                    dst_ref=ag_ref.at[src],
                    send_sem=ag_send.at[src],
                    recv_sem=ag_recv.at[src],
                    device_id=(src,),
                    device_id_type=pl.DeviceIdType.MESH,
                ).wait_recv()
            for s in range(N_DEV):
                out_ref[s * seg:(s + 1) * seg] = ag_ref[s]

        with _scope("drain_sends"):
            for r in rs_rdmas:
                r.wait_send()
            for r in ag_rdmas:
                r.wait_send()

    return pl.pallas_call(
        body,
        out_shape=jax.ShapeDtypeStruct((n, d), WIRE_DT),
        in_specs=[
            pl.BlockSpec(memory_space=pl.ANY),
            pl.BlockSpec(memory_space=pltpu.VMEM),
        ],
        out_specs=pl.BlockSpec(memory_space=pltpu.VMEM),
        scratch_shapes=[
            pltpu.VMEM((m_per, d), jnp.float32),
            pltpu.VMEM((n, d), WIRE_DT),
            pltpu.VMEM((seg, d), WIRE_DT),
            pltpu.VMEM((N_DEV, seg, d), WIRE_DT),
            pltpu.VMEM((N_DEV, seg, d), WIRE_DT),
            pltpu.SemaphoreType.DMA((N_DEV,)),
            pltpu.SemaphoreType.DMA((N_DEV,)),
            pltpu.SemaphoreType.DMA((N_DEV,)),
            pltpu.SemaphoreType.DMA((N_DEV,)),
            pltpu.SemaphoreType.DMA,
        ],
        compiler_params=pltpu.CompilerParams(collective_id=0),
    )(table, idx)
